# baseline (speedup 1.0000x reference)
import jax
import jax.numpy as jnp
from jax.experimental import pallas as pl
from jax.experimental.pallas import tpu as pltpu


def _copy_kernel(x_hbm, o_hbm, sems):
    B = x_hbm.shape[0]
    for b in range(B):
        pltpu.make_async_copy(x_hbm.at[b], o_hbm.at[b], sems.at[b]).start()
    for b in range(B):
        pltpu.make_async_copy(x_hbm.at[b], o_hbm.at[b], sems.at[b]).wait()


def kernel(x_img, x_tab, w1, b1, w2, b2):
    B, C, D, H, W = x_img.shape
    S = D * H * W
    x3 = x_img.reshape(B, C, S)
    out = pl.pallas_call(
        _copy_kernel,
        out_shape=jax.ShapeDtypeStruct((B, C, S), x_img.dtype),
        in_specs=[pl.BlockSpec(memory_space=pltpu.MemorySpace.HBM)],
        out_specs=pl.BlockSpec(memory_space=pltpu.MemorySpace.HBM),
        scratch_shapes=[pltpu.SemaphoreType.DMA((B,))],
    )(x3)
    return out.reshape(B, C, D, H, W)


# speedup vs baseline: 9.6471x; 9.6471x over previous
import jax
import jax.numpy as jnp
from jax.experimental import pallas as pl
from jax.experimental.pallas import tpu as pltpu


def _daft_manual_kernel(x_hbm, xt_ref, w1t_ref, b1_ref, w2t_ref, b2_ref,
                        o_hbm, in_bufs, out_bufs, in_sems, out_sems):
    B, C, S = x_hbm.shape

    def start_in(b):
        pltpu.make_async_copy(x_hbm.at[b], in_bufs.at[b % 3],
                              in_sems.at[b % 3]).start()

    def wait_in(b):
        pltpu.make_async_copy(x_hbm.at[b], in_bufs.at[b % 3],
                              in_sems.at[b % 3]).wait()

    def start_out(b):
        pltpu.make_async_copy(out_bufs.at[b % 2], o_hbm.at[b],
                              out_sems.at[b % 2]).start()

    def wait_out(b):
        pltpu.make_async_copy(out_bufs.at[b % 2], o_hbm.at[b],
                              out_sems.at[b % 2]).wait()

    start_in(0)
    start_in(1)

    def body(b, _):
        @pl.when(b + 2 < B)
        def _():
            start_in(b + 2)
        wait_in(b)
        x = in_bufs[b % 3]
        pooled = jnp.sum(x, axis=1, keepdims=True) * (1.0 / S)
        lane = jax.lax.broadcasted_iota(jnp.int32, xt_ref.shape, 1)
        xt_col = jnp.sum(jnp.where(lane == b, xt_ref[...], 0.0),
                         axis=1, keepdims=True)
        z = jnp.concatenate([pooled, xt_col], axis=0)
        h = jax.lax.dot_general(w1t_ref[...], z, (((1,), (0,)), ((), ())),
                                preferred_element_type=jnp.float32)
        h = jnp.maximum(h + b1_ref[...], 0.0)
        y = jax.lax.dot_general(w2t_ref[...], h, (((1,), (0,)), ((), ())),
                                preferred_element_type=jnp.float32)
        y = y + b2_ref[...]
        @pl.when(b >= 2)
        def _():
            wait_out(b)
        out_bufs[b % 2] = y[:C, :] * x + y[C:, :]
        start_out(b)
        return ()

    jax.lax.fori_loop(0, B, body, ())
    wait_out(B - 2)
    wait_out(B - 1)


def kernel(x_img, x_tab, w1, b1, w2, b2):
    B, C, D, H, W = x_img.shape
    S = D * H * W
    P = x_tab.shape[1]
    hidden = w1.shape[1]

    x3 = x_img.reshape(B, C, S)
    xt = x_tab.astype(jnp.float32).T
    w1t = w1.astype(jnp.float32).T
    b1c = b1.astype(jnp.float32).reshape(hidden, 1)
    w2t = w2.astype(jnp.float32).T
    b2c = b2.astype(jnp.float32).reshape(2 * C, 1)

    out = pl.pallas_call(
        _daft_manual_kernel,
        out_shape=jax.ShapeDtypeStruct((B, C, S), x_img.dtype),
        in_specs=[
            pl.BlockSpec(memory_space=pltpu.MemorySpace.HBM),
            pl.BlockSpec((P, B), lambda: (0, 0)),
            pl.BlockSpec((hidden, C + P), lambda: (0, 0)),
            pl.BlockSpec((hidden, 1), lambda: (0, 0)),
            pl.BlockSpec((2 * C, hidden), lambda: (0, 0)),
            pl.BlockSpec((2 * C, 1), lambda: (0, 0)),
        ],
        out_specs=pl.BlockSpec(memory_space=pltpu.MemorySpace.HBM),
        scratch_shapes=[
            pltpu.VMEM((3, C, S), jnp.float32),
            pltpu.VMEM((2, C, S), jnp.float32),
            pltpu.SemaphoreType.DMA((3,)),
            pltpu.SemaphoreType.DMA((2,)),
        ],
    )(x3, xt, w1t, b1c, w2t, b2c)

    return out.reshape(B, C, D, H, W)


# speedup vs baseline: 9.6918x; 1.0046x over previous
import jax
import jax.numpy as jnp
from jax.experimental import pallas as pl
from jax.experimental.pallas import tpu as pltpu


def _daft_manual_kernel(x_hbm, xt_ref, w1t_ref, b1_ref, w2t_ref, b2_ref,
                        o_hbm, in_bufs, out_bufs, in_sems, out_sems):
    B, C, S = x_hbm.shape

    def start_in(b):
        pltpu.make_async_copy(x_hbm.at[b], in_bufs.at[b % 4],
                              in_sems.at[b % 4]).start()

    def wait_in(b):
        pltpu.make_async_copy(x_hbm.at[b], in_bufs.at[b % 4],
                              in_sems.at[b % 4]).wait()

    def start_out(b):
        pltpu.make_async_copy(out_bufs.at[b % 2], o_hbm.at[b],
                              out_sems.at[b % 2]).start()

    def wait_out(b):
        pltpu.make_async_copy(out_bufs.at[b % 2], o_hbm.at[b],
                              out_sems.at[b % 2]).wait()

    start_in(0)
    start_in(1)
    start_in(2)

    def body(b, _):
        @pl.when(b + 3 < B)
        def _():
            start_in(b + 3)
        wait_in(b)
        x = in_bufs[b % 4]
        pooled = jnp.sum(x, axis=1, keepdims=True) * (1.0 / S)
        lane = jax.lax.broadcasted_iota(jnp.int32, xt_ref.shape, 1)
        xt_col = jnp.sum(jnp.where(lane == b, xt_ref[...], 0.0),
                         axis=1, keepdims=True)
        z = jnp.concatenate([pooled, xt_col], axis=0)
        h = jax.lax.dot_general(w1t_ref[...], z, (((1,), (0,)), ((), ())),
                                preferred_element_type=jnp.float32)
        h = jnp.maximum(h + b1_ref[...], 0.0)
        y = jax.lax.dot_general(w2t_ref[...], h, (((1,), (0,)), ((), ())),
                                preferred_element_type=jnp.float32)
        y = y + b2_ref[...]
        @pl.when(b >= 2)
        def _():
            wait_out(b)
        out_bufs[b % 2] = y[:C, :] * x + y[C:, :]
        start_out(b)
        return ()

    jax.lax.fori_loop(0, B, body, ())
    wait_out(B - 2)
    wait_out(B - 1)


def kernel(x_img, x_tab, w1, b1, w2, b2):
    B, C, D, H, W = x_img.shape
    S = D * H * W
    P = x_tab.shape[1]
    hidden = w1.shape[1]

    x3 = x_img.reshape(B, C, S)
    xt = x_tab.astype(jnp.float32).T
    w1t = w1.astype(jnp.float32).T
    b1c = b1.astype(jnp.float32).reshape(hidden, 1)
    w2t = w2.astype(jnp.float32).T
    b2c = b2.astype(jnp.float32).reshape(2 * C, 1)

    out = pl.pallas_call(
        _daft_manual_kernel,
        out_shape=jax.ShapeDtypeStruct((B, C, S), x_img.dtype),
        in_specs=[
            pl.BlockSpec(memory_space=pltpu.MemorySpace.HBM),
            pl.BlockSpec((P, B), lambda: (0, 0)),
            pl.BlockSpec((hidden, C + P), lambda: (0, 0)),
            pl.BlockSpec((hidden, 1), lambda: (0, 0)),
            pl.BlockSpec((2 * C, hidden), lambda: (0, 0)),
            pl.BlockSpec((2 * C, 1), lambda: (0, 0)),
        ],
        out_specs=pl.BlockSpec(memory_space=pltpu.MemorySpace.HBM),
        scratch_shapes=[
            pltpu.VMEM((4, C, S), jnp.float32),
            pltpu.VMEM((2, C, S), jnp.float32),
            pltpu.SemaphoreType.DMA((4,)),
            pltpu.SemaphoreType.DMA((2,)),
        ],
    )(x3, xt, w1t, b1c, w2t, b2c)

    return out.reshape(B, C, D, H, W)


# speedup vs baseline: 9.7404x; 1.0050x over previous
import jax
import jax.numpy as jnp
from jax.experimental import pallas as pl
from jax.experimental.pallas import tpu as pltpu


def _daft_manual_kernel(x_hbm, xt_ref, w1t_ref, b1_ref, w2t_ref, b2_ref,
                        o_hbm, in_bufs, out_bufs, in_sems, out_sems):
    B, C, S = x_hbm.shape

    def start_in(b):
        pltpu.make_async_copy(x_hbm.at[b], in_bufs.at[b % 4],
                              in_sems.at[b % 4]).start()

    def wait_in(b):
        pltpu.make_async_copy(x_hbm.at[b], in_bufs.at[b % 4],
                              in_sems.at[b % 4]).wait()

    def start_out(b):
        pltpu.make_async_copy(out_bufs.at[b % 3], o_hbm.at[b],
                              out_sems.at[b % 3]).start()

    def wait_out(b):
        pltpu.make_async_copy(out_bufs.at[b % 3], o_hbm.at[b],
                              out_sems.at[b % 3]).wait()

    start_in(0)
    start_in(1)
    start_in(2)

    def body(b, _):
        @pl.when(b + 3 < B)
        def _():
            start_in(b + 3)
        wait_in(b)
        x = in_bufs[b % 4]
        pooled = jnp.sum(x, axis=1, keepdims=True) * (1.0 / S)
        lane = jax.lax.broadcasted_iota(jnp.int32, xt_ref.shape, 1)
        xt_col = jnp.sum(jnp.where(lane == b, xt_ref[...], 0.0),
                         axis=1, keepdims=True)
        z = jnp.concatenate([pooled, xt_col], axis=0)
        h = jax.lax.dot_general(w1t_ref[...], z, (((1,), (0,)), ((), ())),
                                preferred_element_type=jnp.float32)
        h = jnp.maximum(h + b1_ref[...], 0.0)
        y = jax.lax.dot_general(w2t_ref[...], h, (((1,), (0,)), ((), ())),
                                preferred_element_type=jnp.float32)
        y = y + b2_ref[...]
        @pl.when(b >= 3)
        def _():
            wait_out(b)
        out_bufs[b % 3] = y[:C, :] * x + y[C:, :]
        start_out(b)
        return ()

    jax.lax.fori_loop(0, B, body, ())
    wait_out(B - 3)
    wait_out(B - 2)
    wait_out(B - 1)


def kernel(x_img, x_tab, w1, b1, w2, b2):
    B, C, D, H, W = x_img.shape
    S = D * H * W
    P = x_tab.shape[1]
    hidden = w1.shape[1]

    x3 = x_img.reshape(B, C, S)
    xt = x_tab.astype(jnp.float32).T
    w1t = w1.astype(jnp.float32).T
    b1c = b1.astype(jnp.float32).reshape(hidden, 1)
    w2t = w2.astype(jnp.float32).T
    b2c = b2.astype(jnp.float32).reshape(2 * C, 1)

    out = pl.pallas_call(
        _daft_manual_kernel,
        out_shape=jax.ShapeDtypeStruct((B, C, S), x_img.dtype),
        in_specs=[
            pl.BlockSpec(memory_space=pltpu.MemorySpace.HBM),
            pl.BlockSpec((P, B), lambda: (0, 0)),
            pl.BlockSpec((hidden, C + P), lambda: (0, 0)),
            pl.BlockSpec((hidden, 1), lambda: (0, 0)),
            pl.BlockSpec((2 * C, hidden), lambda: (0, 0)),
            pl.BlockSpec((2 * C, 1), lambda: (0, 0)),
        ],
        out_specs=pl.BlockSpec(memory_space=pltpu.MemorySpace.HBM),
        scratch_shapes=[
            pltpu.VMEM((4, C, S), jnp.float32),
            pltpu.VMEM((3, C, S), jnp.float32),
            pltpu.SemaphoreType.DMA((4,)),
            pltpu.SemaphoreType.DMA((3,)),
        ],
    )(x3, xt, w1t, b1c, w2t, b2c)

    return out.reshape(B, C, D, H, W)
